# baseline (speedup 1.0000x reference)
import functools

import jax
import jax.numpy as jnp
from jax import lax
from jax.experimental import pallas as pl
from jax.experimental.pallas import tpu as pltpu
from jax.experimental.pallas import tpu_sc as plsc

N = 10000
E = 320000
D = 128
HD = 64
NLAYERS = 3
NC = 2
NS = 16
CHUNK = 128
NCHUNK = 157
EDGES_PAD = NS * NCHUNK * CHUNK
NPAD = 10112
ROWS_PER = NPAD // NS

BLK = 2000
NBLK = N // BLK


def _sc_propagate(x_halves, src_p, dst_p, zeros_rows):
  mesh = plsc.VectorSubcoreMesh(core_axis_name="c", subcore_axis_name="s")

  @functools.partial(
      pl.kernel,
      out_type=jax.ShapeDtypeStruct((NLAYERS, NC, NPAD, HD), jnp.float32),
      mesh=mesh,
      compiler_params=pltpu.CompilerParams(use_tc_tiling_on_sc=False),
      scratch_types=[
          pltpu.VMEM((NCHUNK, CHUNK), jnp.int32),
          pltpu.VMEM((NCHUNK, CHUNK), jnp.int32),
          pltpu.VMEM((CHUNK, HD), jnp.float32),
          pltpu.VMEM_SHARED((NPAD, HD), jnp.float32),
          pltpu.SemaphoreType.DMA,
      ],
  )
  def sc_kernel(x_hbm, src_hbm, dst_hbm, zeros_hbm, out_hbm,
                src_v, dst_v, rows_v, accum, sem):
    c = lax.axis_index("c")
    s = lax.axis_index("s")
    row0 = s * ROWS_PER
    pltpu.sync_copy(src_hbm.at[s], src_v)
    pltpu.sync_copy(dst_hbm.at[s], dst_v)
    pltpu.sync_copy(zeros_hbm.at[pl.ds(row0, ROWS_PER)],
                    accum.at[pl.ds(row0, ROWS_PER)])
    plsc.subcore_barrier()

    for layer in range(NLAYERS):
      if layer == 0:
        table = x_hbm.at[c]
      else:
        table = out_hbm.at[layer - 1, c]

      def chunk_body(j, carry):
        pltpu.async_copy(table.at[src_v.at[j]], rows_v, sem).wait()
        pltpu.sync_copy(rows_v, accum.at[dst_v.at[j]], add=True)
        return carry

      lax.fori_loop(0, NCHUNK, chunk_body, 0)
      plsc.subcore_barrier()
      pltpu.sync_copy(accum.at[pl.ds(row0, ROWS_PER)],
                      out_hbm.at[layer, c, pl.ds(row0, ROWS_PER)])
      if layer < NLAYERS - 1:
        pltpu.sync_copy(zeros_hbm.at[pl.ds(row0, ROWS_PER)],
                        accum.at[pl.ds(row0, ROWS_PER)])
      plsc.subcore_barrier()

  return sc_kernel(x_halves, src_p, dst_p, zeros_rows)


def _gru_body(m_ref, x_ref, wl_ref, wh_ref, whh_ref, bih_ref, bhh_ref,
              out_ref):
  wl = wl_ref[...]
  wh = wh_ref[...]
  whh = whh_ref[...]
  bih = bih_ref[...]
  bhh = bhh_ref[...]
  dn = (((1,), (1,)), ((), ()))
  h = jnp.zeros((BLK, D), jnp.float32)
  seq = [(m_ref[2, 0], m_ref[2, 1]),
         (m_ref[1, 0], m_ref[1, 1]),
         (m_ref[0, 0], m_ref[0, 1]),
         (x_ref[0], x_ref[1])]
  for lo, hi in seq:
    gi = (lax.dot_general(lo, wl, dn, preferred_element_type=jnp.float32)
          + lax.dot_general(hi, wh, dn, preferred_element_type=jnp.float32)
          + bih)
    gh = lax.dot_general(h, whh, dn, preferred_element_type=jnp.float32) + bhh
    r = jax.nn.sigmoid(gi[:, :D] + gh[:, :D])
    z = jax.nn.sigmoid(gi[:, D:2 * D] + gh[:, D:2 * D])
    n = jnp.tanh(gi[:, 2 * D:] + r * gh[:, 2 * D:])
    h = (1.0 - z) * n + z * h
  out_ref[...] = h


def _tc_gru(msgs, x_halves, W_ih, W_hh, b_ih, b_hh):
  wl = W_ih[:, :HD]
  wh = W_ih[:, HD:]
  bih = b_ih.reshape(1, 3 * D)
  bhh = b_hh.reshape(1, 3 * D)
  return pl.pallas_call(
      _gru_body,
      grid=(NBLK,),
      in_specs=[
          pl.BlockSpec((NLAYERS, NC, BLK, HD), lambda i: (0, 0, i, 0)),
          pl.BlockSpec((NC, BLK, HD), lambda i: (0, i, 0)),
          pl.BlockSpec((3 * D, HD), lambda i: (0, 0)),
          pl.BlockSpec((3 * D, HD), lambda i: (0, 0)),
          pl.BlockSpec((3 * D, D), lambda i: (0, 0)),
          pl.BlockSpec((1, 3 * D), lambda i: (0, 0)),
          pl.BlockSpec((1, 3 * D), lambda i: (0, 0)),
      ],
      out_specs=pl.BlockSpec((BLK, D), lambda i: (i, 0)),
      out_shape=jax.ShapeDtypeStruct((N, D), jnp.float32),
  )(msgs, x_halves, wl, wh, W_hh, bih, bhh)


def kernel(x, ei, W_ih, W_hh, b_ih, b_hh):
  src = ei[0].astype(jnp.int32)
  dst = ei[1].astype(jnp.int32)
  pad = EDGES_PAD - E
  trash = N + (jnp.arange(pad, dtype=jnp.int32) % (NPAD - N))
  src_p = jnp.concatenate([src, jnp.zeros((pad,), jnp.int32)])
  dst_p = jnp.concatenate([dst, trash])
  src_p = src_p.reshape(NS, NCHUNK, CHUNK)
  dst_p = dst_p.reshape(NS, NCHUNK, CHUNK)
  x_halves = jnp.stack([x[:, :HD], x[:, HD:]])
  zeros_rows = jnp.zeros((NPAD, HD), jnp.float32)
  msgs = _sc_propagate(x_halves, src_p, dst_p, zeros_rows)
  return _tc_gru(msgs, x_halves, W_ih, W_hh, b_ih, b_hh)


# speedup vs baseline: 1.0051x; 1.0051x over previous
import functools

import jax
import jax.numpy as jnp
from jax import lax
from jax.experimental import pallas as pl
from jax.experimental.pallas import tpu as pltpu
from jax.experimental.pallas import tpu_sc as plsc

N = 10000
E = 320000
D = 128
HD = 64
NLAYERS = 3
NC = 2
NS = 16
CHUNK = 128
NCHUNK = 157
EDGES_PAD = NS * NCHUNK * CHUNK
NPAD = 10112
ROWS_PER = NPAD // NS

BLK = 2000
NBLK = N // BLK


def _sc_propagate(x_halves, src_p, dst_p, zeros_rows):
  mesh = plsc.VectorSubcoreMesh(core_axis_name="c", subcore_axis_name="s")

  @functools.partial(
      pl.kernel,
      out_type=jax.ShapeDtypeStruct((NLAYERS, NC, NPAD, HD), jnp.float32),
      mesh=mesh,
      compiler_params=pltpu.CompilerParams(use_tc_tiling_on_sc=False),
      scratch_types=[
          pltpu.VMEM((NCHUNK, CHUNK), jnp.int32),
          pltpu.VMEM((NCHUNK, CHUNK), jnp.int32),
          pltpu.VMEM((CHUNK, HD), jnp.float32),
          pltpu.VMEM_SHARED((NPAD, HD), jnp.float32),
          pltpu.SemaphoreType.DMA,
      ],
  )
  def sc_kernel(x_hbm, src_hbm, dst_hbm, zeros_hbm, out_hbm,
                src_v, dst_v, rows_v, accum, sem):
    c = lax.axis_index("c")
    s = lax.axis_index("s")
    row0 = s * ROWS_PER
    pltpu.sync_copy(src_hbm.at[s], src_v)
    pltpu.sync_copy(dst_hbm.at[s], dst_v)
    pltpu.sync_copy(zeros_hbm.at[pl.ds(row0, ROWS_PER)],
                    accum.at[pl.ds(row0, ROWS_PER)])
    plsc.subcore_barrier()

    for layer in range(NLAYERS):
      if layer == 0:
        table = x_hbm.at[c]
      else:
        table = out_hbm.at[layer - 1, c]

      @plsc.parallel_loop(0, NCHUNK, 1, unroll=2)
      def chunk_body(j):
        pltpu.async_copy(table.at[src_v.at[j]], rows_v, sem).wait()
        pltpu.sync_copy(rows_v, accum.at[dst_v.at[j]], add=True)

      plsc.subcore_barrier()
      pltpu.sync_copy(accum.at[pl.ds(row0, ROWS_PER)],
                      out_hbm.at[layer, c, pl.ds(row0, ROWS_PER)])
      if layer < NLAYERS - 1:
        pltpu.sync_copy(zeros_hbm.at[pl.ds(row0, ROWS_PER)],
                        accum.at[pl.ds(row0, ROWS_PER)])
      plsc.subcore_barrier()

  return sc_kernel(x_halves, src_p, dst_p, zeros_rows)


def _gru_body(m_ref, x_ref, wl_ref, wh_ref, whh_ref, bih_ref, bhh_ref,
              out_ref):
  wl = wl_ref[...]
  wh = wh_ref[...]
  whh = whh_ref[...]
  bih = bih_ref[...]
  bhh = bhh_ref[...]
  dn = (((1,), (1,)), ((), ()))
  h = jnp.zeros((BLK, D), jnp.float32)
  seq = [(m_ref[2, 0], m_ref[2, 1]),
         (m_ref[1, 0], m_ref[1, 1]),
         (m_ref[0, 0], m_ref[0, 1]),
         (x_ref[0], x_ref[1])]
  for lo, hi in seq:
    gi = (lax.dot_general(lo, wl, dn, preferred_element_type=jnp.float32)
          + lax.dot_general(hi, wh, dn, preferred_element_type=jnp.float32)
          + bih)
    gh = lax.dot_general(h, whh, dn, preferred_element_type=jnp.float32) + bhh
    r = jax.nn.sigmoid(gi[:, :D] + gh[:, :D])
    z = jax.nn.sigmoid(gi[:, D:2 * D] + gh[:, D:2 * D])
    n = jnp.tanh(gi[:, 2 * D:] + r * gh[:, 2 * D:])
    h = (1.0 - z) * n + z * h
  out_ref[...] = h


def _tc_gru(msgs, x_halves, W_ih, W_hh, b_ih, b_hh):
  wl = W_ih[:, :HD]
  wh = W_ih[:, HD:]
  bih = b_ih.reshape(1, 3 * D)
  bhh = b_hh.reshape(1, 3 * D)
  return pl.pallas_call(
      _gru_body,
      grid=(NBLK,),
      in_specs=[
          pl.BlockSpec((NLAYERS, NC, BLK, HD), lambda i: (0, 0, i, 0)),
          pl.BlockSpec((NC, BLK, HD), lambda i: (0, i, 0)),
          pl.BlockSpec((3 * D, HD), lambda i: (0, 0)),
          pl.BlockSpec((3 * D, HD), lambda i: (0, 0)),
          pl.BlockSpec((3 * D, D), lambda i: (0, 0)),
          pl.BlockSpec((1, 3 * D), lambda i: (0, 0)),
          pl.BlockSpec((1, 3 * D), lambda i: (0, 0)),
      ],
      out_specs=pl.BlockSpec((BLK, D), lambda i: (i, 0)),
      out_shape=jax.ShapeDtypeStruct((N, D), jnp.float32),
  )(msgs, x_halves, wl, wh, W_hh, bih, bhh)


def kernel(x, ei, W_ih, W_hh, b_ih, b_hh):
  src = ei[0].astype(jnp.int32)
  dst = ei[1].astype(jnp.int32)
  pad = EDGES_PAD - E
  trash = N + (jnp.arange(pad, dtype=jnp.int32) % (NPAD - N))
  src_p = jnp.concatenate([src, jnp.zeros((pad,), jnp.int32)])
  dst_p = jnp.concatenate([dst, trash])
  src_p = src_p.reshape(NS, NCHUNK, CHUNK)
  dst_p = dst_p.reshape(NS, NCHUNK, CHUNK)
  x_halves = jnp.stack([x[:, :HD], x[:, HD:]])
  zeros_rows = jnp.zeros((NPAD, HD), jnp.float32)
  msgs = _sc_propagate(x_halves, src_p, dst_p, zeros_rows)
  return _tc_gru(msgs, x_halves, W_ih, W_hh, b_ih, b_hh)
